# baseline (speedup 1.0000x reference)
import functools

import jax
import jax.numpy as jnp
from jax import lax
from jax.experimental import pallas as pl
from jax.experimental.pallas import tpu as pltpu
from jax.experimental.pallas import tpu_sc as plsc

MAX_SEQ = 8192
ATTR = 8
EDIMS = 256

_NC = 2
_NS = 16
_NW = _NC * _NS
_NUNIQ = MAX_SEQ // ATTR
_U = _NUNIQ // _NW
_XCHUNK = MAX_SEQ // _NW

_TC_BLK = 64


def _build_sc():
    mesh = plsc.VectorSubcoreMesh(core_axis_name="c", subcore_axis_name="s")

    @functools.partial(
        pl.kernel,
        mesh=mesh,
        out_type=jax.ShapeDtypeStruct((_NUNIQ, EDIMS), jnp.float32),
        scratch_types=[
            pltpu.VMEM((_U,), jnp.int32),
            pltpu.VMEM((_U, EDIMS), jnp.float32),
            pltpu.SemaphoreType.DMA,
        ],
    )
    def k(x_hbm, table_hbm, out_hbm, uidx_v, rows_v, sem):
        wid = lax.axis_index("s") * _NC + lax.axis_index("c")
        idx_cps = [
            pltpu.async_copy(
                x_hbm.at[jnp.arange(16, dtype=jnp.int32) * ATTR
                         + wid * _XCHUNK + i * 16 * ATTR],
                uidx_v.at[pl.ds(i * 16, 16)], sem)
            for i in range(_U // 16)
        ]
        for cp in idx_cps:
            cp.wait()
        pltpu.async_copy(table_hbm.at[uidx_v], rows_v, sem).wait()
        pltpu.sync_copy(rows_v, out_hbm.at[pl.ds(wid * _U, _U)])

    return k


def _expand_body(c_ref, o_ref):
    o_ref[...] = jnp.broadcast_to(
        c_ref[...][:, None, :], (_TC_BLK, ATTR, EDIMS))


def _expand(compact):
    return pl.pallas_call(
        _expand_body,
        grid=(_NUNIQ // _TC_BLK,),
        in_specs=[pl.BlockSpec((_TC_BLK, EDIMS), lambda i: (i, 0))],
        out_specs=pl.BlockSpec((_TC_BLK, ATTR, EDIMS), lambda i: (i, 0, 0)),
        out_shape=jax.ShapeDtypeStruct((_NUNIQ, ATTR, EDIMS), jnp.float32),
    )(compact)


def kernel(x, E_object_index):
    x = x.astype(jnp.int32)
    compact = _build_sc()(x, E_object_index)
    return _expand(compact).reshape(MAX_SEQ, EDIMS)


# speedup vs baseline: 1.3590x; 1.3590x over previous
import functools

import jax
import jax.numpy as jnp
from jax import lax
from jax.experimental import pallas as pl
from jax.experimental.pallas import tpu as pltpu
from jax.experimental.pallas import tpu_sc as plsc

MAX_SEQ = 8192
ATTR = 8
EDIMS = 256

_NC = 2
_NS = 16
_NW = _NC * _NS
_NUNIQ = MAX_SEQ // ATTR
_U = _NUNIQ // _NW
_XCHUNK = MAX_SEQ // _NW


def _build():
    mesh = plsc.VectorSubcoreMesh(core_axis_name="c", subcore_axis_name="s")

    @functools.partial(
        pl.kernel,
        mesh=mesh,
        out_type=jax.ShapeDtypeStruct((_NUNIQ, ATTR, EDIMS), jnp.float32),
        scratch_types=[
            pltpu.VMEM((_U,), jnp.int32),
            pltpu.VMEM((_U, EDIMS), jnp.float32),
            pltpu.SemaphoreType.DMA,
        ],
    )
    def k(x_hbm, table_hbm, out_hbm, uidx_v, rows_v, sem):
        wid = lax.axis_index("s") * _NC + lax.axis_index("c")
        idx_cps = [
            pltpu.async_copy(
                x_hbm.at[jnp.arange(16, dtype=jnp.int32) * ATTR
                         + wid * _XCHUNK + i * 16 * ATTR],
                uidx_v.at[pl.ds(i * 16, 16)], sem)
            for i in range(_U // 16)
        ]
        for cp in idx_cps:
            cp.wait()
        pltpu.async_copy(table_hbm.at[uidx_v], rows_v, sem).wait()
        out_cps = [
            pltpu.async_copy(rows_v, out_hbm.at[pl.ds(wid * _U, _U), r], sem)
            for r in range(ATTR)
        ]
        for cp in out_cps:
            cp.wait()

    return k


def kernel(x, E_object_index):
    x = x.astype(jnp.int32)
    out3 = _build()(x, E_object_index)
    return out3.reshape(MAX_SEQ, EDIMS)
